# baseline (speedup 1.0000x reference)
import functools

import jax
import jax.numpy as jnp
from jax import lax
from jax.experimental import pallas as pl
from jax.experimental.pallas import tpu as pltpu
from jax.experimental.pallas import tpu_sc as plsc

NC = 2
NS = 16
NW = NC * NS
LANES = 16
BATCH = 16384
EMBED = 64
BPW = BATCH // NW
CHUNK = 128
NCH = BPW // CHUNK

_MESH = dict(core_axis_name="c", subcore_axis_name="s",
             num_cores=NC, num_subcores=NS)
_PARAMS = pltpu.CompilerParams(
    use_tc_tiling_on_sc=False, needs_layout_passes=False)


def _sc_gather_partial(pairs, user_embedding, user_bias,
                       item_embedding, item_bias):
    @functools.partial(
        pl.kernel,
        out_type=(
            jax.ShapeDtypeStruct((NW, LANES), jnp.float32),
            jax.ShapeDtypeStruct((NW, NCH, CHUNK, 1), jnp.float32),
            jax.ShapeDtypeStruct((NW, NCH, CHUNK, 1), jnp.float32),
        ),
        mesh=plsc.VectorSubcoreMesh(**_MESH),
        compiler_params=_PARAMS,
        scratch_types=[
            pltpu.VMEM((2 * BPW,), jnp.int32),
            pltpu.VMEM((NCH, CHUNK), jnp.int32),
            pltpu.VMEM((NCH, CHUNK), jnp.int32),
            pltpu.VMEM((BPW, EMBED), jnp.float32),
            pltpu.VMEM((BPW, EMBED), jnp.float32),
            pltpu.VMEM((NCH, CHUNK, 1), jnp.float32),
            pltpu.VMEM((NCH, CHUNK, 1), jnp.float32),
            pltpu.VMEM((LANES,), jnp.float32),
            pltpu.SemaphoreType.DMA,
            pltpu.SemaphoreType.DMA,
            pltpu.SemaphoreType.DMA,
        ],
    )
    def sc_kernel(pairs_h, uemb_h, ubias_h, iemb_h, ibias_h,
                  parts_h, ubg_h, ibg_h,
                  pairs_v, idxu_v, idxi_v, urows_v, vrows_v, ub_v, ib_v,
                  acc_v, sem_u, sem_v, sem_b):
        wid = lax.axis_index("s") * NC + lax.axis_index("c")
        pltpu.sync_copy(pairs_h.at[wid], pairs_v)
        lane2 = 2 * lax.iota(jnp.int32, LANES)
        for g in range(BPW // LANES):
            off = jnp.int32(2 * LANES * g)
            u = plsc.load_gather(pairs_v, [off + lane2])
            v = plsc.load_gather(pairs_v, [off + lane2 + 1])
            idxu_v[g // 8, pl.ds((g % 8) * LANES, LANES)] = u
            idxi_v[g // 8, pl.ds((g % 8) * LANES, LANES)] = v
        copies = []
        for j in range(NCH):
            copies.append(pltpu.async_copy(
                uemb_h.at[idxu_v.at[j]], urows_v.at[pl.ds(j * CHUNK, CHUNK)],
                sem_u))
            copies.append(pltpu.async_copy(
                iemb_h.at[idxi_v.at[j]], vrows_v.at[pl.ds(j * CHUNK, CHUNK)],
                sem_v))
            copies.append(pltpu.async_copy(
                ubias_h.at[idxu_v.at[j]], ub_v.at[j], sem_b))
            copies.append(pltpu.async_copy(
                ibias_h.at[idxi_v.at[j]], ib_v.at[j], sem_b))
        for c in copies:
            c.wait()
        pltpu.sync_copy(ub_v, ubg_h.at[wid])
        pltpu.sync_copy(ib_v, ibg_h.at[wid])

        def body(i, acc):
            b = i * 4
            for r in range(4):
                for j in range(EMBED // LANES):
                    acc = acc + (urows_v[b + r, pl.ds(j * LANES, LANES)]
                                 * vrows_v[b + r, pl.ds(j * LANES, LANES)])
            return acc

        acc = lax.fori_loop(0, BPW // 4, body,
                            jnp.zeros((LANES,), jnp.float32))
        acc_v[...] = acc
        pltpu.sync_copy(acc_v, parts_h.at[wid])

    return sc_kernel(pairs, user_embedding, user_bias,
                     item_embedding, item_bias)


def _sc_finish(parts, ubg, ibg):
    @functools.partial(
        pl.kernel,
        out_type=jax.ShapeDtypeStruct((NW, BPW), jnp.float32),
        mesh=plsc.VectorSubcoreMesh(**_MESH),
        compiler_params=_PARAMS,
        scratch_types=[
            pltpu.VMEM((NW, LANES), jnp.float32),
            pltpu.VMEM((BPW,), jnp.float32),
            pltpu.VMEM((BPW,), jnp.float32),
            pltpu.VMEM((BPW,), jnp.float32),
        ],
    )
    def fin_kernel(parts_h, ub_h, ib_h, out_h, parts_v, ub_v, ib_v, out_v):
        wid = lax.axis_index("s") * NC + lax.axis_index("c")
        pltpu.sync_copy(parts_h, parts_v)
        pltpu.sync_copy(ub_h.at[wid], ub_v)
        pltpu.sync_copy(ib_h.at[wid], ib_v)
        acc = jnp.zeros((LANES,), jnp.float32)
        for w in range(NW):
            acc = acc + parts_v[w, :]
        s = jnp.sum(acc)
        for g in range(BPW // LANES):
            x = s + ub_v[pl.ds(g * LANES, LANES)] + ib_v[pl.ds(g * LANES, LANES)]
            out_v[pl.ds(g * LANES, LANES)] = 1.0 / (1.0 + jnp.exp(-x))
        pltpu.sync_copy(out_v, out_h.at[wid])

    return fin_kernel(parts, ubg, ibg)


def kernel(inputs, user_embedding, user_bias, item_embedding, item_bias):
    pairs = inputs.reshape(NW, 2 * BPW)
    parts, ubg, ibg = _sc_gather_partial(
        pairs, user_embedding, user_bias, item_embedding, item_bias)
    out = _sc_finish(parts, ubg.reshape(NW, BPW), ibg.reshape(NW, BPW))
    return out.reshape(BATCH, 1)


# speedup vs baseline: 2.3865x; 2.3865x over previous
import functools

import jax
import jax.numpy as jnp
from jax import lax
from jax.experimental import pallas as pl
from jax.experimental.pallas import tpu as pltpu
from jax.experimental.pallas import tpu_sc as plsc

NC = 2
NS = 16
NW = NC * NS
LANES = 16
BATCH = 16384
EMBED = 64
BPW = BATCH // NW
CHUNK = 128
NCH = BPW // CHUNK

_MESH = dict(core_axis_name="c", subcore_axis_name="s",
             num_cores=NC, num_subcores=NS)
_PARAMS = pltpu.CompilerParams(
    use_tc_tiling_on_sc=False, needs_layout_passes=False)


def _sc_gather_partial(pairs, user_embedding, user_bias,
                       item_embedding, item_bias):
    @functools.partial(
        pl.kernel,
        out_type=(
            jax.ShapeDtypeStruct((NW, LANES), jnp.float32),
            jax.ShapeDtypeStruct((NW, NCH, CHUNK), jnp.float32),
            jax.ShapeDtypeStruct((NW, NCH, CHUNK), jnp.float32),
        ),
        mesh=plsc.VectorSubcoreMesh(**_MESH),
        compiler_params=_PARAMS,
        scratch_types=[
            pltpu.VMEM((2 * BPW,), jnp.int32),
            pltpu.VMEM((NCH, CHUNK), jnp.int32),
            pltpu.VMEM((NCH, CHUNK), jnp.int32),
            pltpu.VMEM((BPW, EMBED), jnp.float32),
            pltpu.VMEM((BPW, EMBED), jnp.float32),
            pltpu.VMEM((NCH, CHUNK), jnp.float32),
            pltpu.VMEM((NCH, CHUNK), jnp.float32),
            pltpu.VMEM((LANES,), jnp.float32),
            pltpu.SemaphoreType.DMA,
            pltpu.SemaphoreType.DMA,
            pltpu.SemaphoreType.DMA,
        ],
    )
    def sc_kernel(pairs_h, uemb_h, ubias_h, iemb_h, ibias_h,
                  parts_h, ubg_h, ibg_h,
                  pairs_v, idxu_v, idxi_v, urows_v, vrows_v, ub_v, ib_v,
                  acc_v, sem_u, sem_v, sem_b):
        wid = lax.axis_index("s") * NC + lax.axis_index("c")
        pltpu.sync_copy(pairs_h.at[wid], pairs_v)
        lane2 = 2 * lax.iota(jnp.int32, LANES)
        for g in range(BPW // LANES):
            off = jnp.int32(2 * LANES * g)
            u = plsc.load_gather(pairs_v, [off + lane2])
            v = plsc.load_gather(pairs_v, [off + lane2 + 1])
            idxu_v[g // 8, pl.ds((g % 8) * LANES, LANES)] = u
            idxi_v[g // 8, pl.ds((g % 8) * LANES, LANES)] = v
        copies = []
        for j in range(NCH):
            copies.append(pltpu.async_copy(
                uemb_h.at[idxu_v.at[j]], urows_v.at[pl.ds(j * CHUNK, CHUNK)],
                sem_u))
            copies.append(pltpu.async_copy(
                iemb_h.at[idxi_v.at[j]], vrows_v.at[pl.ds(j * CHUNK, CHUNK)],
                sem_v))
            copies.append(pltpu.async_copy(
                ubias_h.at[idxu_v.at[j]], ub_v.at[j], sem_b))
            copies.append(pltpu.async_copy(
                ibias_h.at[idxi_v.at[j]], ib_v.at[j], sem_b))
        for c in copies:
            c.wait()
        pltpu.sync_copy(ub_v, ubg_h.at[wid])
        pltpu.sync_copy(ib_v, ibg_h.at[wid])

        def body(i, acc):
            b = i * 4
            for r in range(4):
                for j in range(EMBED // LANES):
                    acc = acc + (urows_v[b + r, pl.ds(j * LANES, LANES)]
                                 * vrows_v[b + r, pl.ds(j * LANES, LANES)])
            return acc

        acc = lax.fori_loop(0, BPW // 4, body,
                            jnp.zeros((LANES,), jnp.float32))
        acc_v[...] = acc
        pltpu.sync_copy(acc_v, parts_h.at[wid])

    return sc_kernel(pairs, user_embedding, user_bias,
                     item_embedding, item_bias)


def _sc_finish(parts, ubg, ibg):
    @functools.partial(
        pl.kernel,
        out_type=jax.ShapeDtypeStruct((NW, BPW), jnp.float32),
        mesh=plsc.VectorSubcoreMesh(**_MESH),
        compiler_params=_PARAMS,
        scratch_types=[
            pltpu.VMEM((NW, LANES), jnp.float32),
            pltpu.VMEM((BPW,), jnp.float32),
            pltpu.VMEM((BPW,), jnp.float32),
            pltpu.VMEM((BPW,), jnp.float32),
        ],
    )
    def fin_kernel(parts_h, ub_h, ib_h, out_h, parts_v, ub_v, ib_v, out_v):
        wid = lax.axis_index("s") * NC + lax.axis_index("c")
        pltpu.sync_copy(parts_h, parts_v)
        pltpu.sync_copy(ub_h.at[wid], ub_v)
        pltpu.sync_copy(ib_h.at[wid], ib_v)
        acc = jnp.zeros((LANES,), jnp.float32)
        for w in range(NW):
            acc = acc + parts_v[w, :]
        s = jnp.sum(acc)
        for g in range(BPW // LANES):
            x = s + ub_v[pl.ds(g * LANES, LANES)] + ib_v[pl.ds(g * LANES, LANES)]
            out_v[pl.ds(g * LANES, LANES)] = 1.0 / (1.0 + jnp.exp(-x))
        pltpu.sync_copy(out_v, out_h.at[wid])

    return fin_kernel(parts, ubg, ibg)


def kernel(inputs, user_embedding, user_bias, item_embedding, item_bias):
    pairs = inputs.reshape(NW, 2 * BPW)
    parts, ubg, ibg = _sc_gather_partial(
        pairs, user_embedding, user_bias.reshape(-1),
        item_embedding, item_bias.reshape(-1))
    out = _sc_finish(parts, ubg.reshape(NW, BPW), ibg.reshape(NW, BPW))
    return out.reshape(BATCH, 1)
